# Initial kernel scaffold; baseline (speedup 1.0000x reference)
import jax
import jax.numpy as jnp
from jax.experimental import pallas as pl


def kernel(inst, table):
    raise NotImplementedError("write your pallas kernel here")


# speedup vs baseline: 5.7568x; 5.7568x over previous
import functools

import jax
import jax.numpy as jnp
from jax import lax
from jax.experimental import pallas as pl
from jax.experimental.pallas import tpu as pltpu
from jax.experimental.pallas import tpu_sc as plsc

NC = 2
NS = 16
NW = NC * NS
CHUNK = 512


@functools.cache
def _build(B, V, D):
    b_per_w = B // NW
    n_chunks = b_per_w // CHUNK
    mesh = plsc.VectorSubcoreMesh(
        core_axis_name="c", subcore_axis_name="s",
        num_cores=NC, num_subcores=NS,
    )

    def body(inst_hbm, table_hbm, out_hbm, idx_v, rows_v, sem):
        wid = lax.axis_index("s") * NC + lax.axis_index("c")
        base = wid * b_per_w

        def step(i, carry):
            off = base + i * CHUNK
            pltpu.sync_copy(inst_hbm.at[pl.ds(off, CHUNK)], idx_v)
            pltpu.async_copy(table_hbm.at[idx_v], rows_v, sem).wait()
            pltpu.sync_copy(rows_v, out_hbm.at[pl.ds(off, CHUNK)])
            return carry

        lax.fori_loop(0, n_chunks, step, 0)

    return pl.kernel(
        body,
        out_type=jax.ShapeDtypeStruct((B, D), jnp.float32),
        mesh=mesh,
        compiler_params=pltpu.CompilerParams(use_tc_tiling_on_sc=False),
        scratch_types=[
            pltpu.VMEM((CHUNK,), jnp.int32),
            pltpu.VMEM((CHUNK, D), jnp.float32),
            pltpu.SemaphoreType.DMA,
        ],
    )


def kernel(inst, table):
    R, S = inst.shape
    V, D = table.shape
    B = R * S
    flat = inst.reshape(B)
    out = _build(B, V, D)(flat, table)
    return out.reshape(R, S, D)


# speedup vs baseline: 6.4889x; 1.1272x over previous
import functools

import jax
import jax.numpy as jnp
from jax import lax
from jax.experimental import pallas as pl
from jax.experimental.pallas import tpu as pltpu
from jax.experimental.pallas import tpu_sc as plsc

NC = 2
NS = 16
NW = NC * NS
CHUNK = 512
NBUF = 4


@functools.cache
def _build(B, V, D):
    b_per_w = B // NW
    n_chunks = b_per_w // CHUNK
    assert n_chunks % NBUF == 0
    mesh = plsc.VectorSubcoreMesh(
        core_axis_name="c", subcore_axis_name="s",
        num_cores=NC, num_subcores=NS,
    )

    def body(inst_hbm, table_hbm, out_hbm, idx_v, rows_v, isem, gsem, ssem):
        wid = lax.axis_index("s") * NC + lax.axis_index("c")
        base = wid * b_per_w

        def issue_idx(chunk, b):
            pltpu.async_copy(
                inst_hbm.at[pl.ds(base + chunk * CHUNK, CHUNK)],
                idx_v.at[b], isem.at[b])

        def wait_idx(b):
            pltpu.make_async_copy(
                inst_hbm.at[pl.ds(base, CHUNK)], idx_v.at[b], isem.at[b]
            ).wait()

        def issue_gather(b):
            pltpu.async_copy(
                table_hbm.at[idx_v.at[b]], rows_v.at[b], gsem.at[b])

        def wait_gather(b):
            pltpu.make_async_copy(
                table_hbm.at[idx_v.at[b]], rows_v.at[b], gsem.at[b]
            ).wait()

        def issue_store(chunk, b):
            pltpu.async_copy(
                rows_v.at[b],
                out_hbm.at[pl.ds(base + chunk * CHUNK, CHUNK)], ssem.at[b])

        def wait_store(b):
            pltpu.make_async_copy(
                rows_v.at[b], out_hbm.at[pl.ds(base, CHUNK)], ssem.at[b]
            ).wait()

        for b in range(NBUF):
            issue_idx(b, b)

        @pl.loop(0, n_chunks, step=NBUF)
        def _(g):
            for b in range(NBUF):
                i = g + b
                pb = (b - 1) % NBUF

                @pl.when(i >= NBUF)
                def _():
                    wait_store(b)

                wait_idx(b)
                issue_gather(b)

                @pl.when(i >= 1)
                def _():
                    wait_gather(pb)
                    issue_store(i - 1, pb)

                    @pl.when(i - 1 + NBUF < n_chunks)
                    def _():
                        issue_idx(i - 1 + NBUF, pb)

        lb = (n_chunks - 1) % NBUF
        wait_gather(lb)
        issue_store(n_chunks - 1, lb)
        for b in range(NBUF):
            wait_store(b)

    return pl.kernel(
        body,
        out_type=jax.ShapeDtypeStruct((B, D), jnp.float32),
        mesh=mesh,
        compiler_params=pltpu.CompilerParams(use_tc_tiling_on_sc=False),
        scratch_types=[
            pltpu.VMEM((NBUF, CHUNK), jnp.int32),
            pltpu.VMEM((NBUF, CHUNK, D), jnp.float32),
            pltpu.SemaphoreType.DMA((NBUF,)),
            pltpu.SemaphoreType.DMA((NBUF,)),
            pltpu.SemaphoreType.DMA((NBUF,)),
        ],
    )


def kernel(inst, table):
    R, S = inst.shape
    V, D = table.shape
    B = R * S
    flat = inst.reshape(B)
    out = _build(B, V, D)(flat, table)
    return out.reshape(R, S, D)
